# baseline (speedup 1.0000x reference)
import jax
import jax.numpy as jnp
from jax.experimental import pallas as pl
from jax.experimental.pallas import tpu as pltpu

_NSTREAM = 4


def _se_layer(x, w1, w2):
    b, c, h, w = x.shape
    hid, c_in = w1.shape
    out_ch, hid2 = w2.shape
    assert c_in == c and hid2 == hid

    ns = _NSTREAM if c % _NSTREAM == 0 else 1
    cs = c // ns

    w_pad = max(128, -(-w // 128) * 128)
    slab_bytes = cs * h * w_pad * 4
    budget = 16 << 20
    tb = 1
    for d in range(b, 0, -1):
        if b % d == 0 and ns * d * slab_bytes <= budget:
            tb = d
            break
    nb = b // tb

    def _se_body(*refs):
        x_refs = refs[:ns]
        w1t_ref, w2t_ref, o_ref = refs[ns:]
        parts = [jnp.sum(r[...], axis=(2, 3)) for r in x_refs]
        y = parts[0] if ns == 1 else jnp.concatenate(parts, axis=1)
        hcur = jnp.dot(y, w1t_ref[...], preferred_element_type=jnp.float32)
        hcur = jnp.maximum(hcur, 0.0)
        logits = jnp.dot(hcur, w2t_ref[...],
                         preferred_element_type=jnp.float32)
        m = jnp.max(logits, axis=-1, keepdims=True)
        e = jnp.exp(logits - m)
        probs = e * pl.reciprocal(jnp.sum(e, axis=-1, keepdims=True),
                                  approx=False)
        o_ref[...] = probs[None]

    w1t = jnp.asarray(w1).T * (1.0 / (h * w))
    w2t = jnp.asarray(w2).T

    vmem_limit = min(2 * ns * tb * slab_bytes + (4 << 20), 56 << 20)

    def _x_spec(k):
        return pl.BlockSpec((tb, cs, h, w), lambda i, _k=k: (i, _k, 0, 0))

    out = pl.pallas_call(
        _se_body,
        out_shape=jax.ShapeDtypeStruct((nb, tb, out_ch), jnp.float32),
        grid=(nb,),
        in_specs=[_x_spec(k) for k in range(ns)] + [
            pl.BlockSpec((c, hid), lambda i: (0, 0)),
            pl.BlockSpec((hid, out_ch), lambda i: (0, 0)),
        ],
        out_specs=pl.BlockSpec((1, tb, out_ch), lambda i: (i, 0, 0)),
        compiler_params=pltpu.CompilerParams(
            dimension_semantics=("parallel",),
            vmem_limit_bytes=vmem_limit,
        ),
    )(*([x] * ns), w1t, w2t)

    return out.reshape(b, out_ch, 1, 1)


def kernel(x, w1, w2):
    return _se_layer(x, w1, w2)
